# baseline (speedup 1.0000x reference)
import functools

import jax
import jax.numpy as jnp
from jax import lax
from jax.experimental import pallas as pl
from jax.experimental.pallas import tpu as pltpu
from jax.experimental.pallas import tpu_sc as plsc

_LANES = 16
_G = 4


def _pos_embed_sc(row_embed, col_embed, B, H, W, D):
  info = plsc.get_sparse_core_info()
  NC, NS = info.num_cores, info.num_subcores
  NW = NC * NS
  assert H == NW and B % _G == 0
  mesh = plsc.VectorSubcoreMesh(core_axis_name="c", subcore_axis_name="s")

  @functools.partial(
      pl.kernel,
      mesh=mesh,
      out_type=jax.ShapeDtypeStruct((B, H * W, 2 * D), jnp.float32),
      scratch_types=[
          pltpu.VMEM((D,), jnp.float32),
          pltpu.VMEM((_G, W, 2 * D), jnp.float32),
          pltpu.SemaphoreType.DMA,
      ],
  )
  def k(row_hbm, col_hbm, out_hbm, row_v, block_v, sem):
    wid = lax.axis_index("s") * NC + lax.axis_index("c")
    fills = [pltpu.make_async_copy(row_hbm.at[wid], row_v, sem)]
    for g in range(_G):
      fills.append(
          pltpu.make_async_copy(
              col_hbm.at[pl.ds(0, W), :],
              block_v.at[g, :, pl.ds(D, D)],
              sem,
          )
      )
    for cp in fills:
      cp.start()
    for cp in fills:
      cp.wait()
    for c in range(D // _LANES):
      v = row_v[pl.ds(c * _LANES, _LANES)]
      for g in range(_G):
        for r in range(W):
          block_v[g, r, pl.ds(c * _LANES, _LANES)] = v
    outs = [
        pltpu.make_async_copy(
            block_v,
            out_hbm.at[pl.ds(g * _G, _G), pl.ds(wid * W, W), :],
            sem,
        )
        for g in range(B // _G)
    ]
    for cp in outs:
      cp.start()
    for cp in outs:
      cp.wait()

  return k(row_embed, col_embed)


def kernel(x, row_embed, col_embed):
  B, _, H, W = x.shape
  D = row_embed.shape[-1]
  return _pos_embed_sc(row_embed, col_embed, B, H, W, D)


# speedup vs baseline: 1.2172x; 1.2172x over previous
import functools

import jax
import jax.numpy as jnp
from jax import lax
from jax.experimental import pallas as pl
from jax.experimental.pallas import tpu as pltpu
from jax.experimental.pallas import tpu_sc as plsc

_LANES = 16


def _pos_embed_sc(row_embed, col_embed, B, H, W, D):
  info = plsc.get_sparse_core_info()
  NC, NS = info.num_cores, info.num_subcores
  assert H == NC * NS and B % NC == 0
  mesh = plsc.VectorSubcoreMesh(core_axis_name="c", subcore_axis_name="s")

  @functools.partial(
      pl.kernel,
      mesh=mesh,
      out_type=jax.ShapeDtypeStruct((B, H * W, 2 * D), jnp.float32),
      scratch_types=[
          pltpu.VMEM((2, D), jnp.float32),
          pltpu.VMEM((2 * W, 2 * D), jnp.float32),
          pltpu.SemaphoreType.DMA,
      ],
  )
  def k(row_hbm, col_hbm, out_hbm, rows_v, block_v, sem):
    cid = lax.axis_index("c")
    sid = lax.axis_index("s")
    fills = [
        pltpu.make_async_copy(row_hbm.at[pl.ds(2 * sid, 2), :], rows_v, sem)
    ]
    for o in range(2):
      fills.append(
          pltpu.make_async_copy(
              col_hbm.at[pl.ds(0, W), :],
              block_v.at[pl.ds(o * W, W), pl.ds(D, D)],
              sem,
          )
      )
    for cp in fills:
      cp.start()
    for cp in fills:
      cp.wait()
    for c in range(D // _LANES):
      for o in range(2):
        v = rows_v[o, pl.ds(c * _LANES, _LANES)]
        for r in range(W):
          block_v[o * W + r, pl.ds(c * _LANES, _LANES)] = v
    nb = B // NC
    b0 = cid * nb
    outs = [
        pltpu.make_async_copy(
            block_v,
            out_hbm.at[b0 + b, pl.ds(2 * sid * W, 2 * W), :],
            sem,
        )
        for b in range(nb)
    ]
    for cp in outs:
      cp.start()
    for cp in outs:
      cp.wait()

  return k(row_embed, col_embed)


def kernel(x, row_embed, col_embed):
  B, _, H, W = x.shape
  D = row_embed.shape[-1]
  return _pos_embed_sc(row_embed, col_embed, B, H, W, D)


# speedup vs baseline: 1.3432x; 1.1036x over previous
import functools

import jax
import jax.numpy as jnp
from jax import lax
from jax.experimental import pallas as pl
from jax.experimental.pallas import tpu as pltpu
from jax.experimental.pallas import tpu_sc as plsc

_LANES = 16


def _pos_embed_sc(row_embed, col_embed, B, H, W, D):
  info = plsc.get_sparse_core_info()
  NC, NS = info.num_cores, info.num_subcores
  NW = NC * NS
  assert H == NW and W % 2 == 0
  mesh = plsc.VectorSubcoreMesh(core_axis_name="c", subcore_axis_name="s")

  @functools.partial(
      pl.kernel,
      mesh=mesh,
      out_type=jax.ShapeDtypeStruct((B, H * W, 2 * D), jnp.float32),
      scratch_types=[
          pltpu.VMEM((D,), jnp.float32),
          pltpu.VMEM((W, 2 * D), jnp.float32),
          pltpu.SemaphoreType.DMA,
      ],
  )
  def k(row_hbm, col_hbm, out_hbm, row_v, block_v, sem):
    wid = lax.axis_index("s") * NC + lax.axis_index("c")
    a = pltpu.make_async_copy(row_hbm.at[wid], row_v, sem)
    b = pltpu.make_async_copy(
        col_hbm.at[pl.ds(0, W), :], block_v.at[:, pl.ds(D, D)], sem
    )
    a.start()
    b.start()
    a.wait()
    b.wait()
    for c in range(D // _LANES):
      v = row_v[pl.ds(c * _LANES, _LANES)]
      for r in range(W):
        block_v[r, pl.ds(c * _LANES, _LANES)] = v
    hw = W // 2
    outs = [
        pltpu.make_async_copy(
            block_v.at[pl.ds(half * hw, hw), :],
            out_hbm.at[bb, pl.ds(wid * W + half * hw, hw), :],
            sem,
        )
        for bb in range(B)
        for half in range(2)
    ]
    for cp in outs:
      cp.start()
    for cp in outs:
      cp.wait()

  return k(row_embed, col_embed)


def kernel(x, row_embed, col_embed):
  B, _, H, W = x.shape
  D = row_embed.shape[-1]
  return _pos_embed_sc(row_embed, col_embed, B, H, W, D)
